# baseline (device time: 84772 ns/iter reference)
import jax
import jax.numpy as jnp
from jax import lax
from jax.experimental import pallas as pl
from jax.experimental.pallas import tpu as pltpu

T = 1024
T_LOC = 512
D = 1024
F = 2048
E = 8
E_LOC = 4
CAP = 320
NC = 4
FC = F // NC
NS = 7

_HIGH = lax.Precision.HIGHEST


def _body(
    x_ref, r_ref, w1_hbm, w2_hbm, out_ref,
    xf_ref, rsc_ref, tri_ref, g_ref, w1buf, w2buf, of_ref, sendbuf, comm_ref,
    send_sems, recv_sems, wdma_sems, csend_sem, crecv_sem,
):
    ix = lax.axis_index("x")
    iy = lax.axis_index("y")
    iz = lax.axis_index("z")
    partner = (1 - ix, iy, iz)
    pr = 1 - ix

    barrier = pltpu.get_barrier_semaphore()
    pl.semaphore_signal(
        barrier, inc=1, device_id=partner, device_id_type=pl.DeviceIdType.MESH
    )
    pl.semaphore_wait(barrier, 1)

    rdma_x = pltpu.make_async_remote_copy(
        src_ref=x_ref,
        dst_ref=xf_ref.at[pl.ds(ix * T_LOC, T_LOC)],
        send_sem=send_sems.at[0],
        recv_sem=recv_sems.at[0],
        device_id=partner,
        device_id_type=pl.DeviceIdType.MESH,
    )
    rdma_r = pltpu.make_async_remote_copy(
        src_ref=r_ref,
        dst_ref=rsc_ref.at[ix],
        send_sem=send_sems.at[1],
        recv_sem=recv_sems.at[1],
        device_id=partner,
        device_id_type=pl.DeviceIdType.MESH,
    )
    rdma_x.start()
    rdma_r.start()

    for j in (0, 1):
        pltpu.make_async_copy(
            w1_hbm.at[j], w1buf.at[j], wdma_sems.at[j, 0]
        ).start()
        pltpu.make_async_copy(
            w2_hbm.at[j], w2buf.at[j], wdma_sems.at[j, 1]
        ).start()

    xf_ref[pl.ds(ix * T_LOC, T_LOC), :] = x_ref[:, :]
    rsc_ref[ix] = r_ref[:, :]
    it = lax.broadcasted_iota(jnp.int32, (T, T), 0)
    jt = lax.broadcasted_iota(jnp.int32, (T, T), 1)
    tri_ref[0] = (it <= jt).astype(jnp.float32)
    tri_ref[1] = (it == jt).astype(jnp.float32)

    rdma_r.wait()
    g_loc = jnp.concatenate(
        [
            jnp.dot(x_ref[:, :], rsc_ref[0], preferred_element_type=jnp.float32,
                    precision=_HIGH),
            jnp.dot(x_ref[:, :], rsc_ref[1], preferred_element_type=jnp.float32,
                    precision=_HIGH),
        ],
        axis=1,
    )
    g_ref[pl.ds(ix * T_LOC, T_LOC), :] = g_loc

    rdma_x.wait()
    g_rem = jnp.concatenate(
        [
            jnp.dot(xf_ref[pl.ds(pr * T_LOC, T_LOC), :], rsc_ref[0],
                    preferred_element_type=jnp.float32, precision=_HIGH),
            jnp.dot(xf_ref[pl.ds(pr * T_LOC, T_LOC), :], rsc_ref[1],
                    preferred_element_type=jnp.float32, precision=_HIGH),
        ],
        axis=1,
    )
    g_ref[pl.ds(pr * T_LOC, T_LOC), :] = g_rem

    g = g_ref[:, :]
    lane = lax.broadcasted_iota(jnp.int32, (T, E), 1)
    g1 = jnp.max(g, axis=1, keepdims=True)
    i1 = jnp.min(jnp.where(g == g1, lane, E), axis=1, keepdims=True)
    gm = jnp.where(lane == i1, -1e30, g)
    g2 = jnp.max(gm, axis=1, keepdims=True)
    i2 = jnp.min(jnp.where(gm == g2, lane, E), axis=1, keepdims=True)
    e2 = jnp.exp(g2 - g1)
    denom = 1.0 + e2
    wt1 = 1.0 / denom
    wt2 = e2 / denom

    lane4 = lax.broadcasted_iota(jnp.int32, (T, E_LOC), 1) + ix * E_LOC
    w_all = jnp.where(i1 == lane4, wt1, 0.0) + jnp.where(i2 == lane4, wt2, 0.0)
    sel_all = (w_all > 0.0).astype(jnp.float32)
    rows_w = lax.dot_general(
        w_all, tri_ref[1], (((0,), (0,)), ((), ())),
        preferred_element_type=jnp.float32,
    )
    rows_sel = lax.dot_general(
        sel_all, tri_ref[1], (((0,), (0,)), ((), ())),
        preferred_element_type=jnp.float32,
    )
    rank_rows = lax.dot_general(
        rows_sel, tri_ref[0], (((1,), (0,)), ((), ())),
        preferred_element_type=jnp.float32,
    )
    rows_s = jnp.where(rows_sel > 0.0, rank_rows - 1.0, -1.0).astype(jnp.int32)

    cap_iota = lax.broadcasted_iota(jnp.int32, (CAP, T), 0)
    of = jnp.zeros((T, D), jnp.float32)
    for j in range(E_LOC):
        slot = j % 2
        pltpu.make_async_copy(
            w1_hbm.at[j], w1buf.at[slot], wdma_sems.at[slot, 0]
        ).wait()
        pltpu.make_async_copy(
            w2_hbm.at[j], w2buf.at[slot], wdma_sems.at[slot, 1]
        ).wait()

        w_row = rows_w[j : j + 1, :]
        slot_row = rows_s[j : j + 1, :]
        p = (cap_iota == slot_row).astype(jnp.float32)
        pw = p * w_row
        xsel = jnp.dot(p, xf_ref[:, :], preferred_element_type=jnp.float32)
        h = jnp.maximum(
            jnp.dot(xsel, w1buf[slot], preferred_element_type=jnp.float32), 0.0
        )
        o = jnp.dot(h, w2buf[slot], preferred_element_type=jnp.float32)
        of = of + lax.dot_general(
            pw, o, (((0,), (0,)), ((), ())), preferred_element_type=jnp.float32
        )

        if j + 2 < E_LOC:
            pltpu.make_async_copy(
                w1_hbm.at[j + 2], w1buf.at[slot], wdma_sems.at[slot, 0]
            ).start()
            pltpu.make_async_copy(
                w2_hbm.at[j + 2], w2buf.at[slot], wdma_sems.at[slot, 1]
            ).start()

    of_ref[:, :] = of
    sendbuf[:, :] = of_ref[pl.ds(pr * T_LOC, T_LOC), :].astype(jnp.bfloat16)
    rdma_c = pltpu.make_async_remote_copy(
        src_ref=sendbuf,
        dst_ref=comm_ref,
        send_sem=csend_sem,
        recv_sem=crecv_sem,
        device_id=partner,
        device_id_type=pl.DeviceIdType.MESH,
    )
    rdma_c.start()
    rdma_c.wait()

    out_ref[:, :] = of_ref[pl.ds(ix * T_LOC, T_LOC), :] + comm_ref[
        :, :
    ].astype(jnp.float32)


def kernel(x, router, W1, W2):
    return pl.pallas_call(
        _body,
        out_shape=jax.ShapeDtypeStruct((T_LOC, D), jnp.float32),
        in_specs=[
            pl.BlockSpec(memory_space=pltpu.VMEM),
            pl.BlockSpec(memory_space=pltpu.VMEM),
            pl.BlockSpec(memory_space=pltpu.MemorySpace.HBM),
            pl.BlockSpec(memory_space=pltpu.MemorySpace.HBM),
        ],
        out_specs=pl.BlockSpec(memory_space=pltpu.VMEM),
        scratch_shapes=[
            pltpu.VMEM((T, D), jnp.float32),
            pltpu.VMEM((2, D, E_LOC), jnp.float32),
            pltpu.VMEM((2, T, T), jnp.float32),
            pltpu.VMEM((T, E), jnp.float32),
            pltpu.VMEM((2, D, F), jnp.float32),
            pltpu.VMEM((2, F, D), jnp.float32),
            pltpu.VMEM((T, D), jnp.float32),
            pltpu.VMEM((T_LOC, D), jnp.bfloat16),
            pltpu.VMEM((T_LOC, D), jnp.bfloat16),
            pltpu.SemaphoreType.DMA((2,)),
            pltpu.SemaphoreType.DMA((2,)),
            pltpu.SemaphoreType.DMA((2, 2)),
            pltpu.SemaphoreType.DMA,
            pltpu.SemaphoreType.DMA,
        ],
        compiler_params=pltpu.CompilerParams(
            collective_id=0, vmem_limit_bytes=125 * 1024 * 1024
        ),
    )(x, router, W1, W2)


# device time: 68522 ns/iter; 1.2372x vs baseline; 1.2372x over previous
import jax
import jax.numpy as jnp
from jax import lax
from jax.experimental import pallas as pl
from jax.experimental.pallas import tpu as pltpu

T = 1024
T_LOC = 512
D = 1024
F = 2048
E = 8
E_LOC = 4
CAP = 320
NC = 4
FC = F // NC
NS = 7

_HIGH = lax.Precision.HIGHEST


def _body(
    x_ref, r_ref, w1_hbm, w2_hbm, out_ref,
    xf_ref, rsc_ref, tri_ref, g_ref, w1buf, w2buf, of_ref, sendbuf, comm_ref,
    send_sems, recv_sems, wdma_sems, csend_sem, crecv_sem,
):
    ix = lax.axis_index("x")
    iy = lax.axis_index("y")
    iz = lax.axis_index("z")
    partner = (1 - ix, iy, iz)
    pr = 1 - ix

    barrier = pltpu.get_barrier_semaphore()
    pl.semaphore_signal(
        barrier, inc=1, device_id=partner, device_id_type=pl.DeviceIdType.MESH
    )
    pl.semaphore_wait(barrier, 1)

    rdma_r = pltpu.make_async_remote_copy(
        src_ref=r_ref,
        dst_ref=rsc_ref.at[ix],
        send_sem=send_sems.at[1],
        recv_sem=recv_sems.at[1],
        device_id=partner,
        device_id_type=pl.DeviceIdType.MESH,
    )
    rdma_r.start()
    xf_ref[pl.ds(ix * T_LOC, T_LOC), :] = x_ref[:, :].astype(jnp.bfloat16)
    rdma_x = pltpu.make_async_remote_copy(
        src_ref=xf_ref.at[pl.ds(ix * T_LOC, T_LOC)],
        dst_ref=xf_ref.at[pl.ds(ix * T_LOC, T_LOC)],
        send_sem=send_sems.at[0],
        recv_sem=recv_sems.at[0],
        device_id=partner,
        device_id_type=pl.DeviceIdType.MESH,
    )
    rdma_x.start()

    for j in (0, 1):
        pltpu.make_async_copy(
            w1_hbm.at[j], w1buf.at[j], wdma_sems.at[j, 0]
        ).start()
        pltpu.make_async_copy(
            w2_hbm.at[j], w2buf.at[j], wdma_sems.at[j, 1]
        ).start()

    rsc_ref[ix] = r_ref[:, :]
    it = lax.broadcasted_iota(jnp.int32, (T, T), 0)
    jt = lax.broadcasted_iota(jnp.int32, (T, T), 1)
    tri_ref[0] = (it <= jt).astype(jnp.float32)
    tri_ref[1] = (it == jt).astype(jnp.float32)

    rdma_r.wait()
    g_loc = jnp.concatenate(
        [
            jnp.dot(x_ref[:, :], rsc_ref[0], preferred_element_type=jnp.float32,
                    precision=_HIGH),
            jnp.dot(x_ref[:, :], rsc_ref[1], preferred_element_type=jnp.float32,
                    precision=_HIGH),
        ],
        axis=1,
    )
    g_ref[pl.ds(ix * T_LOC, T_LOC), :] = g_loc
    rdma_g = pltpu.make_async_remote_copy(
        src_ref=g_ref.at[pl.ds(ix * T_LOC, T_LOC)],
        dst_ref=g_ref.at[pl.ds(ix * T_LOC, T_LOC)],
        send_sem=send_sems.at[2],
        recv_sem=recv_sems.at[2],
        device_id=partner,
        device_id_type=pl.DeviceIdType.MESH,
    )
    rdma_g.start()

    rdma_x.wait()
    rdma_g.wait()
    g = g_ref[:, :]
    lane = lax.broadcasted_iota(jnp.int32, (T, E), 1)
    g1 = jnp.max(g, axis=1, keepdims=True)
    i1 = jnp.min(jnp.where(g == g1, lane, E), axis=1, keepdims=True)
    gm = jnp.where(lane == i1, -1e30, g)
    g2 = jnp.max(gm, axis=1, keepdims=True)
    i2 = jnp.min(jnp.where(gm == g2, lane, E), axis=1, keepdims=True)
    e2 = jnp.exp(g2 - g1)
    denom = 1.0 + e2
    wt1 = 1.0 / denom
    wt2 = e2 / denom

    lane4 = lax.broadcasted_iota(jnp.int32, (T, E_LOC), 1) + ix * E_LOC
    w_all = jnp.where(i1 == lane4, wt1, 0.0) + jnp.where(i2 == lane4, wt2, 0.0)
    sel_all = (w_all > 0.0).astype(jnp.float32)
    rows_w = lax.dot_general(
        w_all, tri_ref[1], (((0,), (0,)), ((), ())),
        preferred_element_type=jnp.float32,
    )
    rows_sel = lax.dot_general(
        sel_all, tri_ref[1], (((0,), (0,)), ((), ())),
        preferred_element_type=jnp.float32,
    )
    rank_rows = lax.dot_general(
        rows_sel, tri_ref[0], (((1,), (0,)), ((), ())),
        preferred_element_type=jnp.float32,
    )
    rows_s = jnp.where(rows_sel > 0.0, rank_rows - 1.0, -1.0).astype(jnp.int32)

    cap_iota = lax.broadcasted_iota(jnp.int32, (CAP, T), 0)
    of = jnp.zeros((T, D), jnp.float32)
    for j in range(E_LOC):
        slot = j % 2
        pltpu.make_async_copy(
            w1_hbm.at[j], w1buf.at[slot], wdma_sems.at[slot, 0]
        ).wait()
        pltpu.make_async_copy(
            w2_hbm.at[j], w2buf.at[slot], wdma_sems.at[slot, 1]
        ).wait()

        w_row = rows_w[j : j + 1, :]
        slot_row = rows_s[j : j + 1, :]
        p = (cap_iota == slot_row).astype(jnp.float32)
        pw = p * w_row
        xsel = jnp.dot(
            p.astype(jnp.bfloat16), xf_ref[:, :],
            preferred_element_type=jnp.float32,
        )
        h = jnp.maximum(
            jnp.dot(xsel, w1buf[slot], preferred_element_type=jnp.float32), 0.0
        )
        o = jnp.dot(h, w2buf[slot], preferred_element_type=jnp.float32)
        of = of + lax.dot_general(
            pw, o, (((0,), (0,)), ((), ())), preferred_element_type=jnp.float32
        )

        if j + 2 < E_LOC:
            pltpu.make_async_copy(
                w1_hbm.at[j + 2], w1buf.at[slot], wdma_sems.at[slot, 0]
            ).start()
            pltpu.make_async_copy(
                w2_hbm.at[j + 2], w2buf.at[slot], wdma_sems.at[slot, 1]
            ).start()

    of_ref[:, :] = of
    sendbuf[:, :] = of_ref[pl.ds(pr * T_LOC, T_LOC), :].astype(jnp.bfloat16)
    rdma_c = pltpu.make_async_remote_copy(
        src_ref=sendbuf,
        dst_ref=comm_ref,
        send_sem=csend_sem,
        recv_sem=crecv_sem,
        device_id=partner,
        device_id_type=pl.DeviceIdType.MESH,
    )
    rdma_c.start()
    rdma_c.wait()

    out_ref[:, :] = of_ref[pl.ds(ix * T_LOC, T_LOC), :] + comm_ref[
        :, :
    ].astype(jnp.float32)


def kernel(x, router, W1, W2):
    return pl.pallas_call(
        _body,
        out_shape=jax.ShapeDtypeStruct((T_LOC, D), jnp.float32),
        in_specs=[
            pl.BlockSpec(memory_space=pltpu.VMEM),
            pl.BlockSpec(memory_space=pltpu.VMEM),
            pl.BlockSpec(memory_space=pltpu.MemorySpace.HBM),
            pl.BlockSpec(memory_space=pltpu.MemorySpace.HBM),
        ],
        out_specs=pl.BlockSpec(memory_space=pltpu.VMEM),
        scratch_shapes=[
            pltpu.VMEM((T, D), jnp.bfloat16),
            pltpu.VMEM((2, D, E_LOC), jnp.float32),
            pltpu.VMEM((2, T, T), jnp.float32),
            pltpu.VMEM((T, E), jnp.float32),
            pltpu.VMEM((2, D, F), jnp.float32),
            pltpu.VMEM((2, F, D), jnp.float32),
            pltpu.VMEM((T, D), jnp.float32),
            pltpu.VMEM((T_LOC, D), jnp.bfloat16),
            pltpu.VMEM((T_LOC, D), jnp.bfloat16),
            pltpu.SemaphoreType.DMA((3,)),
            pltpu.SemaphoreType.DMA((3,)),
            pltpu.SemaphoreType.DMA((2, 2)),
            pltpu.SemaphoreType.DMA,
            pltpu.SemaphoreType.DMA,
        ],
        compiler_params=pltpu.CompilerParams(
            collective_id=0, vmem_limit_bytes=125 * 1024 * 1024
        ),
    )(x, router, W1, W2)
